# baseline (speedup 1.0000x reference)
import functools

import jax
import jax.numpy as jnp
from jax import lax
from jax.experimental import pallas as pl
from jax.experimental.pallas import tpu as pltpu
from jax.experimental.pallas import tpu_sc as plsc

BATCH = 16384
HIST = 200
D_FIELD = 16

R_CHUNK = 16
NBUF = 2


@functools.cache
def _build(n_batch, n_vocab):
    info = plsc.get_sparse_core_info()
    nw = info.num_cores * info.num_subcores
    rows_per_w = n_batch // nw
    n_chunks = rows_per_w // R_CHUNK
    n_pairs = n_chunks // NBUF

    mesh = plsc.VectorSubcoreMesh(core_axis_name="c", subcore_axis_name="s")

    @functools.partial(
        pl.kernel,
        mesh=mesh,
        out_type=jax.ShapeDtypeStruct((n_batch, HIST, D_FIELD), jnp.float32),
        scratch_types=[
            pltpu.VMEM((NBUF, R_CHUNK, HIST), jnp.int32),
            pltpu.VMEM((NBUF, R_CHUNK, HIST, D_FIELD), jnp.float32),
            pltpu.SemaphoreType.DMA((NBUF,)),
            pltpu.SemaphoreType.DMA((NBUF,)),
            pltpu.SemaphoreType.DMA((NBUF,)),
        ],
        compiler_params=pltpu.CompilerParams(use_tc_tiling_on_sc=False),
    )
    def gather_kernel(idx_hbm, table_hbm, out_hbm, idx_v, rows_v, sem_i, sem_g, sem_o):
        wid = lax.axis_index("s") * info.num_cores + lax.axis_index("c")
        base = wid * rows_per_w

        for b in range(NBUF):
            pltpu.async_copy(
                idx_hbm.at[pl.ds(base + b * R_CHUNK, R_CHUNK)],
                idx_v.at[b],
                sem_i.at[b],
            )

        def pair_body(p, carry):
            for b in range(NBUF):
                i = p * NBUF + b
                off = base + i * R_CHUNK

                @pl.when(p >= 1)
                def _wait_out():
                    pltpu.make_async_copy(
                        rows_v.at[b], out_hbm.at[pl.ds(off, R_CHUNK)], sem_o.at[b]
                    ).wait()

                pltpu.make_async_copy(
                    idx_hbm.at[pl.ds(off, R_CHUNK)], idx_v.at[b], sem_i.at[b]
                ).wait()

                for r in range(R_CHUNK):
                    pltpu.async_copy(
                        table_hbm.at[idx_v.at[b, r]],
                        rows_v.at[b, r],
                        sem_g.at[b],
                    )
                for r in range(R_CHUNK):
                    pltpu.make_async_copy(
                        table_hbm.at[idx_v.at[b, r]],
                        rows_v.at[b, r],
                        sem_g.at[b],
                    ).wait()

                pltpu.async_copy(
                    rows_v.at[b], out_hbm.at[pl.ds(off, R_CHUNK)], sem_o.at[b]
                )

                @pl.when(p + 1 < n_pairs)
                def _prefetch():
                    pltpu.async_copy(
                        idx_hbm.at[pl.ds(off + NBUF * R_CHUNK, R_CHUNK)],
                        idx_v.at[b],
                        sem_i.at[b],
                    )

            return carry

        lax.fori_loop(0, n_pairs, pair_body, 0)

        for b in range(NBUF):
            pltpu.make_async_copy(
                rows_v.at[b], out_hbm.at[pl.ds(base, R_CHUNK)], sem_o.at[b]
            ).wait()

    return gather_kernel


def kernel(lookup, table):
    return _build(lookup.shape[0], table.shape[0])(lookup.astype(jnp.int32), table)


# speedup vs baseline: 1.8607x; 1.8607x over previous
import functools

import jax
import jax.numpy as jnp
from jax import lax
from jax.experimental import pallas as pl
from jax.experimental.pallas import tpu as pltpu
from jax.experimental.pallas import tpu_sc as plsc

BATCH = 16384
HIST = 200
D_FIELD = 16

BPW = 512
KT = 2
KR = 8
BT = 4


@functools.cache
def _build(n_batch, n_vocab):
    info = plsc.get_sparse_core_info()
    nc = info.num_cores

    mesh = plsc.VectorSubcoreMesh(core_axis_name="c", subcore_axis_name="s")

    @functools.partial(
        pl.kernel,
        mesh=mesh,
        out_type=jax.ShapeDtypeStruct((HIST, KT, 128, KR, 128), jnp.float32),
        scratch_types=[
            pltpu.VMEM((2, BPW), jnp.int32),
            pltpu.VMEM((2, BPW, D_FIELD), jnp.float32),
            pltpu.VMEM((2, KT, BT, KR, 128), jnp.float32),
            pltpu.SemaphoreType.DMA((2,)),
            pltpu.SemaphoreType.DMA((2,)),
            pltpu.SemaphoreType.DMA((2,)),
        ],
        compiler_params=pltpu.CompilerParams(
            use_tc_tiling_on_sc=False, needs_layout_passes=False
        ),
    )
    def gather_kernel(idx_hbm, table_hbm, out_hbm, idx_v, rows_v, stg_v, sem_i, sem_g, sem_o):
        wid = lax.axis_index("s") * nc + lax.axis_index("c")
        bbase = wid * BPW
        lane = lax.iota(jnp.int32, 16)

        pltpu.async_copy(idx_hbm.at[0, pl.ds(bbase, BPW)], idx_v.at[0], sem_i.at[0])
        pltpu.async_copy(idx_hbm.at[1, pl.ds(bbase, BPW)], idx_v.at[1], sem_i.at[1])

        def step(p, carry):
            for s in range(2):
                i = 2 * p + s
                sj = 1 - s

                @pl.when(i < HIST)
                def _fire():
                    pltpu.make_async_copy(
                        idx_hbm.at[0, pl.ds(bbase, BPW)], idx_v.at[s], sem_i.at[s]
                    ).wait()
                    pltpu.async_copy(
                        table_hbm.at[idx_v.at[s]], rows_v.at[s], sem_g.at[s]
                    )

                @pl.when((i >= 1) & (i <= HIST))
                def _finish():
                    j = i - 1
                    pltpu.make_async_copy(
                        table_hbm.at[idx_v.at[sj]], rows_v.at[sj], sem_g.at[sj]
                    ).wait()

                    @pl.when(i + 1 < HIST)
                    def _prefetch_idx():
                        pltpu.async_copy(
                            idx_hbm.at[i + 1, pl.ds(bbase, BPW)],
                            idx_v.at[sj],
                            sem_i.at[sj],
                        )

                    @pl.when(j >= 2)
                    def _wait_out():
                        pltpu.make_async_copy(
                            stg_v.at[sj],
                            out_hbm.at[0, :, pl.ds(wid * BT, BT)],
                            sem_o.at[sj],
                        ).wait()

                    rows2d = rows_v.at[sj]
                    for kt in range(KT):
                        for bt in range(BT):
                            for kr in range(KR):
                                col = jnp.full((16,), kt * KR + kr, jnp.int32)
                                for cb in range(8):
                                    rvec = lane + (bt * 128 + cb * 16)
                                    v = plsc.load_gather(rows2d, [rvec, col])
                                    stg_v[sj, kt, bt, kr, pl.ds(cb * 16, 16)] = v

                    pltpu.async_copy(
                        stg_v.at[sj],
                        out_hbm.at[j, :, pl.ds(wid * BT, BT)],
                        sem_o.at[sj],
                    )

            return carry

        lax.fori_loop(0, HIST // 2 + 1, step, 0)

        for s in range(2):
            pltpu.make_async_copy(
                stg_v.at[s], out_hbm.at[0, :, pl.ds(wid * BT, BT)], sem_o.at[s]
            ).wait()

    return gather_kernel


def kernel(lookup, table):
    idx_t = lookup.T.astype(jnp.int32)
    t5 = _build(lookup.shape[0], table.shape[0])(idx_t, table)
    return (
        t5.transpose(0, 1, 3, 2, 4)
        .reshape(HIST, D_FIELD, BATCH)
        .transpose(2, 0, 1)
    )


# speedup vs baseline: 2.4019x; 1.2908x over previous
import functools

import jax
import jax.numpy as jnp
from jax import lax
from jax.experimental import pallas as pl
from jax.experimental.pallas import tpu as pltpu
from jax.experimental.pallas import tpu_sc as plsc

BATCH = 16384
HIST = 200
D_FIELD = 16

BPW = 512
KT = 2
KR = 8
BT = 4


@functools.cache
def _build(n_batch, n_vocab):
    info = plsc.get_sparse_core_info()
    nc = info.num_cores

    mesh = plsc.VectorSubcoreMesh(core_axis_name="c", subcore_axis_name="s")

    @functools.partial(
        pl.kernel,
        mesh=mesh,
        out_type=jax.ShapeDtypeStruct((HIST, KT, 128, KR, 128), jnp.float32),
        scratch_types=[
            pltpu.VMEM((2, BPW), jnp.int32),
            pltpu.VMEM((2, BPW, D_FIELD), jnp.float32),
            pltpu.VMEM((2, KT, BT, KR, 128), jnp.float32),
            pltpu.SemaphoreType.DMA((2,)),
            pltpu.SemaphoreType.DMA((2,)),
            pltpu.SemaphoreType.DMA((2,)),
        ],
        compiler_params=pltpu.CompilerParams(
            use_tc_tiling_on_sc=False, needs_layout_passes=False
        ),
    )
    def gather_kernel(idx_hbm, table_hbm, out_hbm, idx_v, rows_v, stg_v, sem_i, sem_g, sem_o):
        wid = lax.axis_index("s") * nc + lax.axis_index("c")
        bbase = wid * BPW
        lane = lax.iota(jnp.int32, 16)

        pltpu.async_copy(idx_hbm.at[0, pl.ds(bbase, BPW)], idx_v.at[0], sem_i.at[0])
        pltpu.async_copy(idx_hbm.at[1, pl.ds(bbase, BPW)], idx_v.at[1], sem_i.at[1])

        def step(p, carry):
            for s in range(2):
                i = 2 * p + s
                sj = 1 - s

                @pl.when(i < HIST)
                def _fire():
                    pltpu.make_async_copy(
                        idx_hbm.at[0, pl.ds(bbase, BPW)], idx_v.at[s], sem_i.at[s]
                    ).wait()
                    pltpu.async_copy(
                        table_hbm.at[idx_v.at[s]], rows_v.at[s], sem_g.at[s]
                    )

                @pl.when((i >= 1) & (i <= HIST))
                def _finish():
                    j = i - 1
                    pltpu.make_async_copy(
                        table_hbm.at[idx_v.at[sj]], rows_v.at[sj], sem_g.at[sj]
                    ).wait()

                    @pl.when(i + 1 < HIST)
                    def _prefetch_idx():
                        pltpu.async_copy(
                            idx_hbm.at[i + 1, pl.ds(bbase, BPW)],
                            idx_v.at[sj],
                            sem_i.at[sj],
                        )

                    @pl.when(j >= 2)
                    def _wait_out():
                        pltpu.make_async_copy(
                            stg_v.at[sj],
                            out_hbm.at[0, :, pl.ds(wid * BT, BT)],
                            sem_o.at[sj],
                        ).wait()

                    rows2d = rows_v.at[sj]
                    for kt in range(KT):
                        for bt in range(BT):
                            for kr in range(KR):
                                col = jnp.full((16,), kt * KR + kr, jnp.int32)
                                vs = [
                                    plsc.load_gather(
                                        rows2d,
                                        [lane + (bt * 128 + cb * 16), col],
                                    )
                                    for cb in range(8)
                                ]
                                for cb in range(8):
                                    stg_v[sj, kt, bt, kr, pl.ds(cb * 16, 16)] = (
                                        vs[cb]
                                    )

                    pltpu.async_copy(
                        stg_v.at[sj],
                        out_hbm.at[j, :, pl.ds(wid * BT, BT)],
                        sem_o.at[sj],
                    )

            return carry

        lax.fori_loop(0, HIST // 2 + 1, step, 0)

        for s in range(2):
            pltpu.make_async_copy(
                stg_v.at[s], out_hbm.at[0, :, pl.ds(wid * BT, BT)], sem_o.at[s]
            ).wait()

    return gather_kernel


def kernel(lookup, table):
    idx_t = lookup.T.astype(jnp.int32)
    t5 = _build(lookup.shape[0], table.shape[0])(idx_t, table)
    return (
        t5.transpose(0, 1, 3, 2, 4)
        .reshape(HIST, D_FIELD, BATCH)
        .transpose(2, 0, 1)
    )
